# baseline (speedup 1.0000x reference)
import jax
import jax.numpy as jnp
from jax.experimental import pallas as pl
from jax.experimental.pallas import tpu as pltpu

_N = 2048
_NUM_PHYS = 600000
_THRESH = 1.0
_BLK = 256
_HALF = _N // 2
_CW = 128
_NCHUNK = (_HALF + _BLK) // _CW
_NMASK = _BLK // _CW
_NSTRIP = _N // _BLK


def _notch_kernel(gate_ref, axc_ref, bxc_ref, ayc_ref, byc_ref,
                  axr_ref, bxr_ref, ayr_ref, byr_ref, out_ref,
                  axb_ref, bxb_ref, ayb_ref, byb_ref, wm_ref, acc_ref):
    wide = jnp.zeros((1, _CW), jnp.float32)
    axb_ref[...] = axc_ref[...] + wide
    bxb_ref[...] = bxc_ref[...] + wide
    ayb_ref[...] = ayc_ref[...] + wide
    byb_ref[...] = byc_ref[...] + wide

    lrow = jax.lax.broadcasted_iota(jnp.int32, (_BLK, 2 * _BLK), 0)
    lcol = jax.lax.broadcasted_iota(jnp.int32, (_BLK, 2 * _BLK), 1)
    t = jnp.where(lcol < _BLK, lcol, lcol + (_HALF - _BLK)) - lrow
    wm_ref[...] = jnp.where((t >= 1) & (t < _HALF), 1.0,
                            jnp.where(t == _HALF, 0.5, 0.0)).astype(jnp.float32)
    acc_ref[...] = jnp.zeros((_BLK, _CW), jnp.float32)

    def strip(r, carry):
        base = r * _BLK
        axc = axb_ref[pl.ds(base, _BLK), :]
        bxc = bxb_ref[pl.ds(base, _BLK), :]
        ayc = ayb_ref[pl.ds(base, _BLK), :]
        byc = byb_ref[pl.ds(base, _BLK), :]
        for k in range(_NCHUNK):
            co = base + k * _CW
            axr = axr_ref[:, pl.ds(co, _CW)]
            bxr = bxr_ref[:, pl.ds(co, _CW)]
            ayr = ayr_ref[:, pl.ds(co, _CW)]
            byr = byr_ref[:, pl.ds(co, _CW)]
            dx = jnp.maximum(jnp.maximum(axc - bxr, axr - bxc), 0.0)
            dy = jnp.maximum(jnp.maximum(ayc - byr, ayr - byc), 0.0)
            p = jnp.maximum((_THRESH - dx) - dy, 0.0)
            p2 = p * p
            if k < _NMASK:
                p2 = wm_ref[:, k * _CW:(k + 1) * _CW] * p2
            elif k >= _NCHUNK - _NMASK:
                kk = k - (_NCHUNK - 2 * _NMASK)
                p2 = wm_ref[:, kk * _CW:(kk + 1) * _CW] * p2
            acc_ref[...] += p2
        return carry

    jax.lax.fori_loop(0, _NSTRIP, strip, jnp.int32(0))
    out_ref[0, 0] = jnp.sum(acc_ref[...]) * gate_ref[0, 0]


def kernel(pos, macro_mask, macro_size_x, macro_size_y):
    x = jax.lax.slice(pos, (0,), (_N,))
    y = jax.lax.slice(pos, (_NUM_PHYS,), (_NUM_PHYS + _N,))
    m = macro_mask
    neg = jnp.where(m, jnp.float32(0.0), jnp.float32(-1e7))
    hx = macro_size_x.astype(jnp.float32) * 0.5 + neg
    hy = macro_size_y.astype(jnp.float32) * 0.5 + neg
    ax, bx = x - hx, x + hx
    ay, by = y - hy, y + hy
    count = jnp.sum(m.astype(jnp.int32))
    gate = jnp.where(count < 2, 0.0, 1.0).astype(jnp.float32).reshape(1, 1)

    col = lambda v: v.reshape(_N, 1)
    dbl = lambda v: jnp.concatenate([v, v]).reshape(1, 2 * _N)

    out = pl.pallas_call(
        _notch_kernel,
        in_specs=[
            pl.BlockSpec(memory_space=pltpu.SMEM),
            pl.BlockSpec((_N, 1), lambda: (0, 0)),
            pl.BlockSpec((_N, 1), lambda: (0, 0)),
            pl.BlockSpec((_N, 1), lambda: (0, 0)),
            pl.BlockSpec((_N, 1), lambda: (0, 0)),
            pl.BlockSpec((1, 2 * _N), lambda: (0, 0)),
            pl.BlockSpec((1, 2 * _N), lambda: (0, 0)),
            pl.BlockSpec((1, 2 * _N), lambda: (0, 0)),
            pl.BlockSpec((1, 2 * _N), lambda: (0, 0)),
        ],
        out_shape=jax.ShapeDtypeStruct((1, 1), jnp.float32),
        out_specs=pl.BlockSpec(memory_space=pltpu.SMEM),
        scratch_shapes=[
            pltpu.VMEM((_N, _CW), jnp.float32),
            pltpu.VMEM((_N, _CW), jnp.float32),
            pltpu.VMEM((_N, _CW), jnp.float32),
            pltpu.VMEM((_N, _CW), jnp.float32),
            pltpu.VMEM((_BLK, 2 * _BLK), jnp.float32),
            pltpu.VMEM((_BLK, _CW), jnp.float32),
        ],
    )(gate, col(ax), col(bx), col(ay), col(by),
      dbl(ax), dbl(bx), dbl(ay), dbl(by))

    return out.reshape(())


# speedup vs baseline: 1.5585x; 1.5585x over previous
import jax
import jax.numpy as jnp
from jax.experimental import pallas as pl
from jax.experimental.pallas import tpu as pltpu

_N = 2048
_NUM_PHYS = 600000
_THRESH = 1.0
_BLK = 256
_HALF = _N // 2
_MID = _HALF - _BLK
_NSTRIP = _N // _BLK


def _notch_kernel(gate_ref, pk_ref, out_ref, rd_ref):
    pk = pk_ref[...]
    rd_ref[:, 0:_N] = pk
    rd_ref[:, _N:2 * _N] = pk

    lrow = jax.lax.broadcasted_iota(jnp.int32, (_BLK, _BLK), 0)
    lcol = jax.lax.broadcasted_iota(jnp.int32, (_BLK, _BLK), 1)
    upper = lcol > lrow
    wlast = jnp.where(lcol < lrow, 1.0,
                      jnp.where(lcol == lrow, 0.5, 0.0)).astype(jnp.float32)

    def strip(r, acc):
        base = r * _BLK
        cT = jnp.transpose(pk_ref[:, pl.ds(base, _BLK)])
        axc = cT[:, 0:1]
        bxc = cT[:, 1:2]
        ayc = cT[:, 2:3]
        byc = cT[:, 3:4]

        def p2(co, w):
            axr = rd_ref[0:1, pl.ds(co, w)]
            bxr = rd_ref[1:2, pl.ds(co, w)]
            ayr = rd_ref[2:3, pl.ds(co, w)]
            byr = rd_ref[3:4, pl.ds(co, w)]
            dx = jnp.maximum(jnp.maximum(axc - bxr, axr - bxc), 0.0)
            dy = jnp.maximum(jnp.maximum(ayc - byr, ayr - byc), 0.0)
            p = jnp.maximum((_THRESH - dx) - dy, 0.0)
            return p * p

        s = jnp.sum(jnp.where(upper, p2(base, _BLK), 0.0))
        s += jnp.sum(p2(base + _BLK, _MID))
        s += jnp.sum(wlast * p2(base + _HALF, _BLK))
        return acc + s

    total = jax.lax.fori_loop(0, _NSTRIP, strip, jnp.float32(0.0))
    out_ref[0, 0] = total * gate_ref[0, 0]


def kernel(pos, macro_mask, macro_size_x, macro_size_y):
    x = jax.lax.slice(pos, (0,), (_N,))
    y = jax.lax.slice(pos, (_NUM_PHYS,), (_NUM_PHYS + _N,))
    m = macro_mask
    neg = jnp.where(m, jnp.float32(0.0), jnp.float32(-1e7))
    hx = macro_size_x.astype(jnp.float32) * 0.5 + neg
    hy = macro_size_y.astype(jnp.float32) * 0.5 + neg
    packed = jnp.concatenate([
        (x - hx).reshape(1, _N), (x + hx).reshape(1, _N),
        (y - hy).reshape(1, _N), (y + hy).reshape(1, _N),
        jnp.zeros((4, _N), jnp.float32)], axis=0)
    count = jnp.sum(m.astype(jnp.int32))
    gate = jnp.where(count < 2, 0.0, 1.0).astype(jnp.float32).reshape(1, 1)

    out = pl.pallas_call(
        _notch_kernel,
        in_specs=[
            pl.BlockSpec(memory_space=pltpu.SMEM),
            pl.BlockSpec((8, _N), lambda: (0, 0)),
        ],
        out_shape=jax.ShapeDtypeStruct((1, 1), jnp.float32),
        out_specs=pl.BlockSpec(memory_space=pltpu.SMEM),
        scratch_shapes=[
            pltpu.VMEM((8, 2 * _N), jnp.float32),
        ],
    )(gate, packed)

    return out.reshape(())
